# baseline (speedup 1.0000x reference)
import dataclasses
import functools

import jax
import jax.numpy as jnp
from jax.experimental import pallas as pl
from jax.experimental.pallas import tpu as pltpu
from jax.experimental.pallas import tpu_sc as plsc

_LANES = 16
_W = 128
_K = 2


def _embedding_gather(wide_table, flat_ids, dim):
    num_groups = wide_table.shape[0]
    nblocks = flat_ids.shape[0]
    rows_per_blk = _K * _W
    num_indices = nblocks * rows_per_blk
    mesh = plsc.VectorSubcoreMesh(core_axis_name="core", subcore_axis_name="subcore")
    cp = pltpu.CompilerParams()
    if "needs_layout_passes" in pltpu.CompilerParams.__dataclass_fields__:
        cp = dataclasses.replace(cp, needs_layout_passes=False)

    @pl.kernel(
        compiler_params=cp,
        out_type=jax.ShapeDtypeStruct((num_indices, dim), wide_table.dtype),
        mesh=mesh,
        scratch_types=[
            pltpu.VMEM((_K, _W), jnp.int32),
            pltpu.VMEM((rows_per_blk,), jnp.int32),
            pltpu.VMEM((rows_per_blk, 4 * dim), jnp.float32),
        ],
    )
    def kernel_fn(table_hbm, idx_hbm, out_hbm, gidx_v, colb_v, buf_v):
        def body(idx_vmem, out_vmem):
            for k in range(_K):

                @pl.loop(0, _W, step=_LANES)
                def _(c, k=k):
                    chunk = idx_vmem[0, k, pl.ds(c, _LANES)]
                    gidx_v[k, pl.ds(c, _LANES)] = jax.lax.shift_right_logical(
                        chunk, 2
                    )
                    colb_v[pl.ds(k * _W + c, _LANES)] = (chunk & 3) * dim

                pltpu.sync_copy(
                    table_hbm.at[gidx_v.at[k]],
                    buf_v.at[pl.ds(k * _W, _W)],
                )

            iota = jax.lax.iota(jnp.int32, _LANES)

            @pl.loop(0, rows_per_blk, step=_LANES)
            def _(j):
                rows = j + iota
                colb = colb_v[pl.ds(j, _LANES)]
                for ci in range(dim):
                    vals = plsc.load_gather(buf_v, [rows, colb + ci])
                    plsc.store_scatter(
                        out_vmem, [rows, jnp.full((_LANES,), ci, jnp.int32)], vals
                    )

        pltpu.emit_pipeline(
            body,
            grid=(nblocks,),
            in_specs=[pl.BlockSpec((1, _K, _W), index_map=lambda i: (i, 0, 0))],
            out_specs=[
                pl.BlockSpec((rows_per_blk, dim), index_map=lambda i: (i, 0))
            ],
            core_axis_name=("core", "subcore"),
            dimension_semantics=(pltpu.PARALLEL,),
        )(idx_hbm, out_hbm)

    return kernel_fn(wide_table, flat_ids)


def kernel(token_ids, weight):
    batch, hist = token_ids.shape
    num_rows, dim = weight.shape
    num_indices = batch * hist
    rows_per_blk = _K * _W
    wide_table = weight.reshape(num_rows // 4, 4 * dim)
    flat_ids = token_ids.reshape(num_indices // rows_per_blk, _K, _W)
    rows = _embedding_gather(wide_table, flat_ids, dim)
    return rows.reshape(batch, hist, dim)


# speedup vs baseline: 1.3287x; 1.3287x over previous
import dataclasses
import functools

import jax
import jax.numpy as jnp
from jax.experimental import pallas as pl
from jax.experimental.pallas import tpu as pltpu
from jax.experimental.pallas import tpu_sc as plsc

_LANES = 16
_BROWS = 4
_HIST = 50
_BLK = _BROWS * _HIST
_BLK_PAD = 208
_NCHUNK = _BLK_PAD // _LANES
_GATHERS = ((0, 104), (104, 96))


def _embedding_gather(wide_table, ids_pad, batch, dim):
    nblocks = ids_pad.shape[0]
    mesh = plsc.VectorSubcoreMesh(core_axis_name="core", subcore_axis_name="subcore")
    cp = pltpu.CompilerParams()
    if "needs_layout_passes" in pltpu.CompilerParams.__dataclass_fields__:
        cp = dataclasses.replace(cp, needs_layout_passes=False)

    @pl.kernel(
        compiler_params=cp,
        out_type=jax.ShapeDtypeStruct((batch, _HIST, dim), wide_table.dtype),
        mesh=mesh,
        scratch_types=[
            pltpu.VMEM((_BLK_PAD,), jnp.int32),
            pltpu.VMEM((_BLK_PAD,), jnp.int32),
            pltpu.VMEM((_BLK_PAD,), jnp.int32),
            pltpu.VMEM((_BLK_PAD,), jnp.int32),
            pltpu.VMEM((_BLK_PAD, 4 * dim), jnp.float32),
            pltpu.SemaphoreType.DMA,
        ],
    )
    def kernel_fn(table_hbm, idx_hbm, out_hbm, gidx_v, colb_v, rb_v, rt_v, buf_v, sem):
        iota = jax.lax.iota(jnp.int32, _LANES)

        @pl.loop(0, _BLK_PAD, step=_LANES)
        def _(j):
            rows = j + iota
            b = rows // _HIST
            rb_v[pl.ds(j, _LANES)] = b
            rt_v[pl.ds(j, _LANES)] = rows - b * _HIST

        def body(idx_vmem, out_vmem):
            @pl.loop(0, _BLK_PAD, step=_LANES)
            def _(c):
                chunk = idx_vmem[0, pl.ds(c, _LANES)]
                gidx_v[pl.ds(c, _LANES)] = jax.lax.shift_right_logical(chunk, 2)
                colb_v[pl.ds(c, _LANES)] = (chunk & 3) * dim

            copies = [
                pltpu.async_copy(
                    table_hbm.at[gidx_v.at[pl.ds(off, num)]],
                    buf_v.at[pl.ds(off, num)],
                    sem,
                )
                for off, num in _GATHERS
            ]
            for c in copies:
                c.wait()

            for cc in range(_NCHUNK):
                j = cc * _LANES
                valid = _BLK - j
                mask = None if valid >= _LANES else iota < valid
                rows = j + iota
                colb = colb_v[pl.ds(j, _LANES)]
                rb = rb_v[pl.ds(j, _LANES)]
                rt = rt_v[pl.ds(j, _LANES)]
                for ci in range(dim):
                    vals = plsc.load_gather(buf_v, [rows, colb + ci], mask=mask)
                    plsc.store_scatter(
                        out_vmem,
                        [rb, rt, jnp.full((_LANES,), ci, jnp.int32)],
                        vals,
                        mask=mask,
                    )

        pltpu.emit_pipeline(
            body,
            grid=(nblocks,),
            in_specs=[pl.BlockSpec((1, _BLK_PAD), index_map=lambda i: (i, 0))],
            out_specs=[
                pl.BlockSpec((_BROWS, _HIST, dim), index_map=lambda i: (i, 0, 0))
            ],
            core_axis_name=("core", "subcore"),
            dimension_semantics=(pltpu.PARALLEL,),
        )(idx_hbm, out_hbm)

    return kernel_fn(wide_table, ids_pad)


def kernel(token_ids, weight):
    batch, hist = token_ids.shape
    num_rows, dim = weight.shape
    wide_table = weight.reshape(num_rows // 4, 4 * dim)
    ids_blk = token_ids.reshape(batch // _BROWS, _BLK)
    ids_pad = jnp.pad(ids_blk, ((0, 0), (0, _BLK_PAD - _BLK)))
    return _embedding_gather(wide_table, ids_pad, batch, dim)


# speedup vs baseline: 1.5386x; 1.1580x over previous
import dataclasses
import functools

import jax
import jax.numpy as jnp
from jax import lax
from jax.experimental import pallas as pl
from jax.experimental.pallas import tpu as pltpu
from jax.experimental.pallas import tpu_sc as plsc

_LANES = 16
_HIST = 50
_BROWS = 4
_BLK = _BROWS * _HIST
_BLK_PAD = 208
_NCHUNK = _BLK_PAD // _LANES
_GATHERS = ((0, 128), (128, 72))
_RING = 2


def _embedding_gather(wide_table, ids_pad, batch, dim):
    nblocks = ids_pad.shape[0]
    mesh = plsc.VectorSubcoreMesh(core_axis_name="core", subcore_axis_name="subcore")
    cp = pltpu.CompilerParams()
    if "needs_layout_passes" in pltpu.CompilerParams.__dataclass_fields__:
        cp = dataclasses.replace(cp, needs_layout_passes=False)

    @pl.kernel(
        compiler_params=cp,
        out_type=jax.ShapeDtypeStruct((batch, _HIST, dim), jnp.float32),
        mesh=mesh,
        scratch_types=(
            [pltpu.VMEM((_BLK_PAD,), jnp.int32) for _ in range(_RING)]
            + [pltpu.VMEM((_BLK_PAD,), jnp.int32) for _ in range(_RING)]
            + [pltpu.VMEM((_BLK_PAD, 4 * dim), jnp.float32) for _ in range(_RING)]
            + [
                pltpu.VMEM((_BLK_PAD,), jnp.int32),
                pltpu.VMEM((_BLK_PAD,), jnp.int32),
                pltpu.SemaphoreType.DMA((_RING,)),
                pltpu.SMEM((1,), jnp.int32),
            ]
        ),
    )
    def kernel_fn(table_hbm, idx_hbm, out_hbm, *scratch):
        gidx = scratch[:_RING]
        colb = scratch[_RING : 2 * _RING]
        bufs = scratch[2 * _RING : 3 * _RING]
        rb_v, rt_v, g_sem, cnt = scratch[3 * _RING :]
        iota = jax.lax.iota(jnp.int32, _LANES)
        cnt[0] = 0
        steps = nblocks // 32

        @pl.loop(0, _BLK_PAD, step=_LANES)
        def _(j):
            rows = j + iota
            b = rows // _HIST
            rb_v[pl.ds(j, _LANES)] = b
            rt_v[pl.ds(j, _LANES)] = rows - b * _HIST

        def idx_math(idx_ref, slot):
            @pl.loop(0, _BLK_PAD, step=_LANES)
            def _(c):
                chunk = idx_ref[0, pl.ds(c, _LANES)]
                gidx[slot][pl.ds(c, _LANES)] = jax.lax.shift_right_logical(
                    chunk, 2
                )
                colb[slot][pl.ds(c, _LANES)] = (chunk & 3) * dim

        def fire(slot):
            for off, num in _GATHERS:
                pltpu.async_copy(
                    table_hbm.at[gidx[slot].at[pl.ds(off, num)]],
                    bufs[slot].at[pl.ds(off, num)],
                    g_sem.at[slot],
                )

        def wait(slot):
            for off, num in _GATHERS:
                pltpu.make_async_copy(
                    table_hbm.at[gidx[slot].at[pl.ds(off, num)]],
                    bufs[slot].at[pl.ds(off, num)],
                    g_sem.at[slot],
                ).wait()

        def extract(out_vmem, slot):
            for cc in range(_NCHUNK):
                j = cc * _LANES
                valid = _BLK - j
                mask = None if valid >= _LANES else iota < valid
                rows = j + iota
                cb = colb[slot][pl.ds(j, _LANES)]
                rb = rb_v[pl.ds(j, _LANES)]
                rt = rt_v[pl.ds(j, _LANES)]
                for ci in range(dim):
                    vals = plsc.load_gather(bufs[slot], [rows, cb + ci], mask=mask)
                    plsc.store_scatter(
                        out_vmem,
                        [rb, rt, jnp.full((_LANES,), ci, jnp.int32)],
                        vals,
                        mask=mask,
                    )

        def body(idx_cur, idx_nxt, out_vmem):
            g = cnt[0]

            @pl.when(g == 0)
            def _():
                idx_math(idx_cur, 0)
                fire(0)

            for slot in range(_RING):

                @pl.when(lax.rem(g, _RING) == slot)
                def _(slot=slot):
                    nxt = (slot + 1) % _RING

                    @pl.when(g < steps - 1)
                    def _():
                        idx_math(idx_nxt, nxt)
                        fire(nxt)

                    wait(slot)
                    extract(out_vmem, slot)

            cnt[0] = g + 1

        pltpu.emit_pipeline(
            body,
            grid=(nblocks,),
            in_specs=[
                pl.BlockSpec((1, _BLK_PAD), index_map=lambda i: (i, 0)),
                pl.BlockSpec(
                    (1, _BLK_PAD),
                    index_map=lambda i: (jnp.minimum(i + 1, nblocks - 1), 0),
                ),
            ],
            out_specs=[
                pl.BlockSpec((_BROWS, _HIST, dim), index_map=lambda i: (i, 0, 0))
            ],
            core_axis_name=("core", "subcore"),
            dimension_semantics=(pltpu.PARALLEL,),
        )(idx_hbm, idx_hbm, out_hbm)

    return kernel_fn(wide_table, ids_pad)


def kernel(token_ids, weight):
    batch, hist = token_ids.shape
    num_rows, dim = weight.shape
    wide_table = weight.reshape(num_rows // 4, 4 * dim)
    ids_blk = token_ids.reshape(batch // _BROWS, _BLK)
    ids_pad = jnp.pad(ids_blk, ((0, 0), (0, _BLK_PAD - _BLK)))
    return _embedding_gather(wide_table, ids_pad, batch, dim)
